# initial kernel scaffold (unmeasured)
import jax
import jax.numpy as jnp
from jax import lax
from jax.experimental import pallas as pl
from jax.experimental.pallas import tpu as pltpu


def kernel(
    x,
):
    def body(*refs):
        pass

    out_shape = jax.ShapeDtypeStruct(..., jnp.float32)
    return pl.pallas_call(body, out_shape=out_shape)(...)


# baseline (device time: 11973 ns/iter reference)
import jax
import jax.numpy as jnp
from jax import lax
from jax.experimental import pallas as pl
from jax.experimental.pallas import tpu as pltpu

N_Y = 4


def kernel(x):
    m, n = x.shape
    blk = m
    assert n == N_Y * blk

    def body(x_ref, out_ref, send_buf, recv_buf, send_sems, recv_sems):
        my_x = lax.axis_index("x")
        my_y = lax.axis_index("y")
        my_z = lax.axis_index("z")

        barrier_sem = pltpu.get_barrier_semaphore()

        for j in range(N_Y):
            send_buf[j] = x_ref[:, j * blk : (j + 1) * blk].astype(jnp.bfloat16)

        for i in range(N_Y):

            @pl.when(my_y == i)
            def _(i=i):
                for j in range(N_Y):
                    if j == i:
                        continue
                    pl.semaphore_signal(
                        barrier_sem,
                        inc=1,
                        device_id=(my_x, j, my_z),
                        device_id_type=pl.DeviceIdType.MESH,
                    )
                pl.semaphore_wait(barrier_sem, N_Y - 1)

                for j in range(N_Y):
                    if j == i:
                        continue
                    rdma = pltpu.make_async_remote_copy(
                        src_ref=send_buf.at[j],
                        dst_ref=recv_buf.at[i],
                        send_sem=send_sems.at[j],
                        recv_sem=recv_sems.at[i],
                        device_id=(my_x, j, my_z),
                        device_id_type=pl.DeviceIdType.MESH,
                    )
                    rdma.start()

                out_ref[i * blk : (i + 1) * blk, :] = send_buf[i]

                for s in range(N_Y):
                    if s == i:
                        continue
                    recv = pltpu.make_async_remote_copy(
                        src_ref=send_buf.at[s],
                        dst_ref=recv_buf.at[s],
                        send_sem=send_sems.at[s],
                        recv_sem=recv_sems.at[s],
                        device_id=(my_x, s, my_z),
                        device_id_type=pl.DeviceIdType.MESH,
                    )
                    recv.wait_recv()
                    out_ref[s * blk : (s + 1) * blk, :] = recv_buf[s]

                for j in range(N_Y):
                    if j == i:
                        continue
                    send = pltpu.make_async_remote_copy(
                        src_ref=send_buf.at[j],
                        dst_ref=recv_buf.at[i],
                        send_sem=send_sems.at[j],
                        recv_sem=recv_sems.at[i],
                        device_id=(my_x, j, my_z),
                        device_id_type=pl.DeviceIdType.MESH,
                    )
                    send.wait_send()

    return pl.pallas_call(
        body,
        out_shape=jax.ShapeDtypeStruct((N_Y * blk, blk), jnp.bfloat16),
        in_specs=[pl.BlockSpec(memory_space=pltpu.VMEM)],
        out_specs=pl.BlockSpec(memory_space=pltpu.VMEM),
        scratch_shapes=[
            pltpu.VMEM((N_Y, blk, blk), jnp.bfloat16),
            pltpu.VMEM((N_Y, blk, blk), jnp.bfloat16),
            pltpu.SemaphoreType.DMA((N_Y,)),
            pltpu.SemaphoreType.DMA((N_Y,)),
        ],
        compiler_params=pltpu.CompilerParams(collective_id=0),
    )(x)


# device time: 11864 ns/iter; 1.0092x vs baseline; 1.0092x over previous
import jax
import jax.numpy as jnp
from jax import lax
from jax.experimental import pallas as pl
from jax.experimental.pallas import tpu as pltpu

N_Y = 4


def kernel(x):
    m, n = x.shape
    blk = m
    assert n == N_Y * blk

    def body(x_ref, out_ref, send_buf, send_sems, recv_sems):
        my_x = lax.axis_index("x")
        my_y = lax.axis_index("y")
        my_z = lax.axis_index("z")

        barrier_sem = pltpu.get_barrier_semaphore()

        for i in range(N_Y):

            @pl.when(my_y == i)
            def _(i=i):
                far_first = sorted(
                    (j for j in range(N_Y) if j != i),
                    key=lambda j: -abs(j - i),
                )
                near_first = far_first[::-1]

                for j in far_first:
                    pl.semaphore_signal(
                        barrier_sem,
                        inc=1,
                        device_id=(my_x, j, my_z),
                        device_id_type=pl.DeviceIdType.MESH,
                    )

                for j in far_first:
                    send_buf[j] = x_ref[:, j * blk : (j + 1) * blk].astype(
                        jnp.bfloat16
                    )

                pl.semaphore_wait(barrier_sem, N_Y - 1)

                for j in far_first:
                    rdma = pltpu.make_async_remote_copy(
                        src_ref=send_buf.at[j],
                        dst_ref=out_ref.at[pl.ds(i * blk, blk)],
                        send_sem=send_sems.at[j],
                        recv_sem=recv_sems.at[i],
                        device_id=(my_x, j, my_z),
                        device_id_type=pl.DeviceIdType.MESH,
                    )
                    rdma.start()

                out_ref[i * blk : (i + 1) * blk, :] = x_ref[
                    :, i * blk : (i + 1) * blk
                ].astype(jnp.bfloat16)

                for s in near_first:
                    recv = pltpu.make_async_remote_copy(
                        src_ref=send_buf.at[s],
                        dst_ref=out_ref.at[pl.ds(s * blk, blk)],
                        send_sem=send_sems.at[s],
                        recv_sem=recv_sems.at[s],
                        device_id=(my_x, s, my_z),
                        device_id_type=pl.DeviceIdType.MESH,
                    )
                    recv.wait_recv()

                for j in near_first:
                    send = pltpu.make_async_remote_copy(
                        src_ref=send_buf.at[j],
                        dst_ref=out_ref.at[pl.ds(i * blk, blk)],
                        send_sem=send_sems.at[j],
                        recv_sem=recv_sems.at[i],
                        device_id=(my_x, j, my_z),
                        device_id_type=pl.DeviceIdType.MESH,
                    )
                    send.wait_send()

    return pl.pallas_call(
        body,
        out_shape=jax.ShapeDtypeStruct((N_Y * blk, blk), jnp.bfloat16),
        in_specs=[pl.BlockSpec(memory_space=pltpu.VMEM)],
        out_specs=pl.BlockSpec(memory_space=pltpu.VMEM),
        scratch_shapes=[
            pltpu.VMEM((N_Y, blk, blk), jnp.bfloat16),
            pltpu.SemaphoreType.DMA((N_Y,)),
            pltpu.SemaphoreType.DMA((N_Y,)),
        ],
        compiler_params=pltpu.CompilerParams(collective_id=0),
    )(x)
